# Initial kernel scaffold; baseline (speedup 1.0000x reference)
import functools
import math

import jax
import jax.numpy as jnp
from jax import lax
from jax.experimental import pallas as pl
from jax.experimental.pallas import tpu as pltpu
from jax.experimental.pallas import tpu_sc as plsc

EMB = 64
LANES = 16
GRP = 128
SCALE = math.sqrt(EMB) / 3.0


@functools.partial(jax.jit, static_argnames=("num_cores", "num_subcores", "chunk"))
def _three_hot_sc(idx_i, idx_v, idx_f, emb_i, emb_v, emb_f,
                  num_cores=2, num_subcores=16, chunk=256):
    n_rows, grp = idx_i.shape
    assert grp == GRP
    B = n_rows * GRP
    NW = num_cores * num_subcores
    per_w = B // NW
    groups = chunk // GRP
    n_chunks = per_w // chunk
    rows_per_w = per_w // GRP

    mesh = plsc.VectorSubcoreMesh(core_axis_name="c", subcore_axis_name="s")

    @functools.partial(
        pl.kernel,
        out_type=jax.ShapeDtypeStruct((B, EMB), jnp.float32),
        mesh=mesh,
        scratch_types=[
            pltpu.VMEM((groups, GRP), jnp.int32),
            pltpu.VMEM((groups, GRP), jnp.int32),
            pltpu.VMEM((groups, GRP), jnp.int32),
            pltpu.VMEM((chunk, EMB), jnp.float32),
            pltpu.VMEM((chunk, EMB), jnp.float32),
            pltpu.VMEM((chunk, EMB), jnp.float32),
            pltpu.SemaphoreType.DMA,
        ],
    )
    def kern(ii, iv, iff, ti, tv, tf, out, xi, xv, xf, bi, bv, bf, sem):
        wid = lax.axis_index("s") * num_cores + lax.axis_index("c")

        def chunk_body(g, _):
            row0 = wid * rows_per_w + g * groups
            tok0 = row0 * GRP
            pltpu.sync_copy(ii.at[pl.ds(row0, groups)], xi)
            pltpu.sync_copy(iv.at[pl.ds(row0, groups)], xv)
            pltpu.sync_copy(iff.at[pl.ds(row0, groups)], xf)
            cps = []
            for j in range(groups):
                d = pl.ds(j * GRP, GRP)
                cps.append(pltpu.async_copy(ti.at[xi.at[j]], bi.at[d], sem))
                cps.append(pltpu.async_copy(tv.at[xv.at[j]], bv.at[d], sem))
                cps.append(pltpu.async_copy(tf.at[xf.at[j]], bf.at[d], sem))
            for c in cps:
                c.wait()

            def row_body(r, _):
                for q in range(EMB // LANES):
                    s = pl.ds(q * LANES, LANES)
                    bi[r, s] = (bi[r, s] + bv[r, s] + bf[r, s]) * SCALE
                return ()

            lax.fori_loop(0, chunk, row_body, ())
            pltpu.sync_copy(bi, out.at[pl.ds(tok0, chunk)])
            return ()

        lax.fori_loop(0, n_chunks, chunk_body, ())

    return kern(idx_i, idx_v, idx_f, emb_i, emb_v, emb_f)


def kernel(tokens, emb_i, emb_v, emb_f):
    lead = tokens.shape[:-1]
    B = tokens.shape[0] * tokens.shape[1]
    t = tokens.reshape(B, 3)
    idx_i = t[:, 0].reshape(B // GRP, GRP)
    idx_v = t[:, 1].reshape(B // GRP, GRP)
    idx_f = t[:, 2].reshape(B // GRP, GRP)
    out = _three_hot_sc(idx_i, idx_v, idx_f, emb_i, emb_v, emb_f)
    return out.reshape(lead + (EMB,))


# speedup vs baseline: 6.2452x; 6.2452x over previous
import functools
import math

import jax
import jax.numpy as jnp
from jax import lax
from jax.experimental import pallas as pl
from jax.experimental.pallas import tpu as pltpu
from jax.experimental.pallas import tpu_sc as plsc

EMB = 64
LANES = 16
GRP = 128
SCALE = math.sqrt(EMB) / 3.0


@functools.partial(jax.jit, static_argnames=("num_cores", "num_subcores", "chunk"))
def _three_hot_sc(idx_i, idx_v, idx_f, emb_i, emb_v, emb_f,
                  num_cores=2, num_subcores=16, chunk=256):
    n_rows, grp = idx_i.shape
    assert grp == GRP
    B = n_rows * GRP
    NW = num_cores * num_subcores
    per_w = B // NW
    groups = chunk // GRP
    n_chunks = per_w // chunk
    rows_per_w = per_w // GRP

    mesh = plsc.VectorSubcoreMesh(core_axis_name="c", subcore_axis_name="s")

    @functools.partial(
        pl.kernel,
        out_type=jax.ShapeDtypeStruct((B, EMB), jnp.float32),
        mesh=mesh,
        compiler_params=pltpu.CompilerParams(use_tc_tiling_on_sc=False),
        scratch_types=[
            pltpu.VMEM((groups, GRP), jnp.int32),
            pltpu.VMEM((groups, GRP), jnp.int32),
            pltpu.VMEM((groups, GRP), jnp.int32),
            pltpu.VMEM((chunk, EMB), jnp.float32),
            pltpu.VMEM((chunk, EMB), jnp.float32),
            pltpu.VMEM((chunk, EMB), jnp.float32),
            pltpu.SemaphoreType.DMA,
        ],
    )
    def kern(ii, iv, iff, ti, tv, tf, out, xi, xv, xf, bi, bv, bf, sem):
        wid = lax.axis_index("s") * num_cores + lax.axis_index("c")

        def chunk_body(g, _):
            row0 = wid * rows_per_w + g * groups
            tok0 = row0 * GRP
            pltpu.sync_copy(ii.at[pl.ds(row0, groups)], xi)
            pltpu.sync_copy(iv.at[pl.ds(row0, groups)], xv)
            pltpu.sync_copy(iff.at[pl.ds(row0, groups)], xf)
            cps = []
            for j in range(groups):
                d = pl.ds(j * GRP, GRP)
                cps.append(pltpu.async_copy(ti.at[xi.at[j]], bi.at[d], sem))
                cps.append(pltpu.async_copy(tv.at[xv.at[j]], bv.at[d], sem))
                cps.append(pltpu.async_copy(tf.at[xf.at[j]], bf.at[d], sem))
            for c in cps:
                c.wait()

            def row_body(r, _):
                for q in range(EMB // LANES):
                    s = pl.ds(q * LANES, LANES)
                    bi[r, s] = (bi[r, s] + bv[r, s] + bf[r, s]) * SCALE
                return ()

            lax.fori_loop(0, chunk, row_body, ())
            pltpu.sync_copy(bi, out.at[pl.ds(tok0, chunk)])
            return ()

        lax.fori_loop(0, n_chunks, chunk_body, ())

    return kern(idx_i, idx_v, idx_f, emb_i, emb_v, emb_f)


def kernel(tokens, emb_i, emb_v, emb_f):
    lead = tokens.shape[:-1]
    B = tokens.shape[0] * tokens.shape[1]
    t = tokens.reshape(B, 3)
    idx_i = t[:, 0].reshape(B // GRP, GRP)
    idx_v = t[:, 1].reshape(B // GRP, GRP)
    idx_f = t[:, 2].reshape(B // GRP, GRP)
    out = _three_hot_sc(idx_i, idx_v, idx_f, emb_i, emb_v, emb_f)
    return out.reshape(lead + (EMB,))
